# baseline (speedup 1.0000x reference)
import functools

import jax
import jax.numpy as jnp
from jax import lax
from jax.experimental import pallas as pl
from jax.experimental.pallas import tpu as pltpu
from jax.experimental.pallas import tpu_sc as plsc

S, E, C = 4096, 16, 512
NW = 32
TPW = S // NW
L = 16
NJ = C // L


def _sc_body(g1_hbm, l1_hbm, g2_hbm, l2_hbm, m1_hbm, m2_hbm, out_hbm,
             g1v, g2v, m1v, m2v, l1v, l2v, outv):
    wid = lax.axis_index("s") * 2 + lax.axis_index("c")
    base = wid * TPW
    pltpu.sync_copy(g1_hbm.at[pl.ds(base, TPW)], g1v.at[pl.ds(0, TPW)])
    pltpu.sync_copy(g2_hbm.at[pl.ds(base, TPW)], g2v.at[pl.ds(0, TPW)])
    pltpu.sync_copy(m1_hbm.at[pl.ds(base, TPW)], m1v)
    pltpu.sync_copy(m2_hbm.at[pl.ds(base, TPW)], m2v)

    def token(i, carry):
        s = base + i
        pltpu.sync_copy(l1_hbm.at[s], l1v)
        pltpu.sync_copy(l2_hbm.at[s], l2v)
        g1 = g1v[pl.ds(i, L)][0]
        g2 = g2v[pl.ds(i, L)][0]
        am = g1 * m1v[i]
        bm = g2 * m2v[i]
        av = [am[e] for e in range(E)]
        bv = [bm[e] for e in range(E)]
        for j in range(NJ):
            l1j = l1v[pl.ds(j * L, L)]
            l2j = l2v[pl.ds(j * L, L)]
            for e in range(E):
                outv[e, pl.ds(j * L, L)] = av[e] * l1j + bv[e] * l2j
        pltpu.sync_copy(outv, out_hbm.at[s])
        return carry

    lax.fori_loop(0, TPW, token, 0)


def kernel(gates1_s, locations1_sc, gates2_s, locations2_sc, mask1_float, mask2_float):
    mesh = plsc.VectorSubcoreMesh(core_axis_name="c", subcore_axis_name="s")
    k = functools.partial(
        pl.kernel,
        out_type=jax.ShapeDtypeStruct((S, E, C), jnp.float32),
        mesh=mesh,
        scratch_types=[
            pltpu.VMEM((TPW + L,), jnp.float32),
            pltpu.VMEM((TPW + L,), jnp.float32),
            pltpu.VMEM((TPW, E), jnp.float32),
            pltpu.VMEM((TPW, E), jnp.float32),
            pltpu.VMEM((C,), jnp.float32),
            pltpu.VMEM((C,), jnp.float32),
            pltpu.VMEM((E, C), jnp.float32),
        ],
    )(_sc_body)
    return k(gates1_s, locations1_sc, gates2_s, locations2_sc,
             mask1_float, mask2_float)


# speedup vs baseline: 1.4449x; 1.4449x over previous
import functools

import jax
import jax.numpy as jnp
from jax import lax
from jax.experimental import pallas as pl
from jax.experimental.pallas import tpu as pltpu
from jax.experimental.pallas import tpu_sc as plsc

S, E, C = 4096, 16, 512
K = 512
NW = 32
L = 16
NJ = C // L
BS = 512
CH = 4


def _sc_body(g1_hbm, l1_hbm, g2_hbm, l2_hbm, m1_hbm, m2_hbm, out_hbm,
             g1v, g2v, m1v, m2v, l1v, l2v, outv):
    tpw = K // NW
    wid = lax.axis_index("s") * 2 + lax.axis_index("c")
    base = wid * tpw
    pltpu.sync_copy(g1_hbm.at[pl.ds(base, tpw)], g1v.at[pl.ds(0, tpw)])
    pltpu.sync_copy(g2_hbm.at[pl.ds(base, tpw)], g2v.at[pl.ds(0, tpw)])
    pltpu.sync_copy(m1_hbm.at[pl.ds(base, tpw)], m1v)
    pltpu.sync_copy(m2_hbm.at[pl.ds(base, tpw)], m2v)

    def token(i, carry):
        s = base + i
        pltpu.sync_copy(l1_hbm.at[s], l1v)
        pltpu.sync_copy(l2_hbm.at[s], l2v)
        g1 = g1v[pl.ds(i, L)][0]
        g2 = g2v[pl.ds(i, L)][0]
        am = g1 * m1v[i]
        bm = g2 * m2v[i]
        av = [am[e] for e in range(E)]
        bv = [bm[e] for e in range(E)]
        for j in range(NJ):
            l1j = l1v[pl.ds(j * L, L)]
            l2j = l2v[pl.ds(j * L, L)]
            for e in range(E):
                outv[e, pl.ds(j * L, L)] = av[e] * l1j + bv[e] * l2j
        pltpu.sync_copy(outv, out_hbm.at[s])
        return carry

    lax.fori_loop(0, tpw, token, 0)


def _sc_combine(g1, l1, g2, l2, m1, m2):
    tpw = K // NW
    mesh = plsc.VectorSubcoreMesh(core_axis_name="c", subcore_axis_name="s")
    k = functools.partial(
        pl.kernel,
        out_type=jax.ShapeDtypeStruct((K, E, C), jnp.float32),
        mesh=mesh,
        scratch_types=[
            pltpu.VMEM((tpw + L,), jnp.float32),
            pltpu.VMEM((tpw + L,), jnp.float32),
            pltpu.VMEM((tpw, E), jnp.float32),
            pltpu.VMEM((tpw, E), jnp.float32),
            pltpu.VMEM((C,), jnp.float32),
            pltpu.VMEM((C,), jnp.float32),
            pltpu.VMEM((E, C), jnp.float32),
        ],
    )(_sc_body)
    return k(g1, l1, g2, l2, m1, m2)


def _tc_body(g1_ref, l1_ref, g2_ref, l2_ref, m1_ref, m2_ref, o_ref):
    g1m1 = (g1_ref[...] * m1_ref[...])[:, :, None]
    g2m2 = (g2_ref[...] * m2_ref[...])[:, :, None]
    for b in range(0, BS, CH):
        sl = slice(b, b + CH)
        l1 = l1_ref[sl][:, None, :]
        l2 = l2_ref[sl][:, None, :]
        o_ref[sl] = g1m1[sl] * l1 + g2m2[sl] * l2


def _tc_combine(g1, l1, g2, l2, m1, m2):
    n = g1.shape[0]
    return pl.pallas_call(
        _tc_body,
        grid=(n // BS,),
        in_specs=[
            pl.BlockSpec((BS, 1), lambda i: (i, 0)),
            pl.BlockSpec((BS, C), lambda i: (i, 0)),
            pl.BlockSpec((BS, 1), lambda i: (i, 0)),
            pl.BlockSpec((BS, C), lambda i: (i, 0)),
            pl.BlockSpec((BS, E), lambda i: (i, 0)),
            pl.BlockSpec((BS, E), lambda i: (i, 0)),
        ],
        out_specs=pl.BlockSpec((BS, E, C), lambda i: (i, 0, 0)),
        out_shape=jax.ShapeDtypeStruct((n, E, C), jnp.float32),
    )(g1.reshape(n, 1), l1, g2.reshape(n, 1), l2, m1, m2)


def kernel(gates1_s, locations1_sc, gates2_s, locations2_sc, mask1_float, mask2_float):
    sc_out = _sc_combine(gates1_s[:K], locations1_sc[:K], gates2_s[:K],
                         locations2_sc[:K], mask1_float[:K], mask2_float[:K])
    tc_out = _tc_combine(gates1_s[K:], locations1_sc[K:], gates2_s[K:],
                         locations2_sc[K:], mask1_float[K:], mask2_float[K:])
    return jnp.concatenate([sc_out, tc_out], axis=0)


# speedup vs baseline: 4.0012x; 2.7692x over previous
import jax
import jax.numpy as jnp
from jax.experimental import pallas as pl
from jax.experimental.pallas import tpu as pltpu

S, E, C = 4096, 16, 512
CT = 128
CH = 4
NBUF = 3
NSTEPS = S // CT


def _body(g1_ref, l1_ref, g2_ref, l2_ref, m1_ref, m2_ref, o_hbm, obuf, sems):
    i = pl.program_id(0)
    slot = jax.lax.rem(i, NBUF)

    @pl.when(i >= NBUF)
    def _():
        pltpu.make_async_copy(
            obuf.at[slot], o_hbm.at[pl.ds((i - NBUF) * CT, CT)], sems.at[slot]
        ).wait()

    g1m1 = (g1_ref[...] * m1_ref[...])[:, :, None]
    g2m2 = (g2_ref[...] * m2_ref[...])[:, :, None]
    for b in range(0, CT, CH):
        sl = slice(b, b + CH)
        l1 = l1_ref[sl][:, None, :]
        l2 = l2_ref[sl][:, None, :]
        obuf[slot, sl] = g1m1[sl] * l1 + g2m2[sl] * l2

    pltpu.make_async_copy(
        obuf.at[slot], o_hbm.at[pl.ds(i * CT, CT)], sems.at[slot]
    ).start()

    @pl.when(i == NSTEPS - 1)
    def _():
        for off in range(NBUF):
            j = NSTEPS - NBUF + off
            pltpu.make_async_copy(
                obuf.at[jax.lax.rem(jnp.int32(j), NBUF)],
                o_hbm.at[pl.ds(j * CT, CT)],
                sems.at[jax.lax.rem(jnp.int32(j), NBUF)],
            ).wait()


def kernel(gates1_s, locations1_sc, gates2_s, locations2_sc, mask1_float, mask2_float):
    g1 = gates1_s.reshape(S, 1)
    g2 = gates2_s.reshape(S, 1)
    return pl.pallas_call(
        _body,
        grid=(NSTEPS,),
        in_specs=[
            pl.BlockSpec((CT, 1), lambda i: (i, 0)),
            pl.BlockSpec((CT, C), lambda i: (i, 0)),
            pl.BlockSpec((CT, 1), lambda i: (i, 0)),
            pl.BlockSpec((CT, C), lambda i: (i, 0)),
            pl.BlockSpec((CT, E), lambda i: (i, 0)),
            pl.BlockSpec((CT, E), lambda i: (i, 0)),
        ],
        out_specs=pl.BlockSpec(memory_space=pl.ANY),
        out_shape=jax.ShapeDtypeStruct((S, E, C), jnp.float32),
        scratch_shapes=[
            pltpu.VMEM((NBUF, CT, E, C), jnp.float32),
            pltpu.SemaphoreType.DMA((NBUF,)),
        ],
    )(g1, locations1_sc, g2, locations2_sc, mask1_float, mask2_float)


# speedup vs baseline: 4.3358x; 1.0836x over previous
import jax
import jax.numpy as jnp
from jax import lax
from jax.experimental import pallas as pl
from jax.experimental.pallas import tpu as pltpu

S, E, C = 4096, 16, 512
CT = 512
QT = 128
CH = 4
NS = 8
NQ = CT // QT
NSTEPS = S // CT
NCHUNKS = S // QT


def _body(g1_ref, l1_ref, g2_ref, l2_ref, m1_ref, m2_ref, o_hbm, obuf, sems):
    i = pl.program_id(0)
    g1m1 = (g1_ref[...] * m1_ref[...])[:, :, None]
    g2m2 = (g2_ref[...] * m2_ref[...])[:, :, None]

    for q in range(NQ):
        k = i * NQ + q
        slot = lax.rem(k, NS)

        @pl.when(k >= NS)
        def _():
            pltpu.make_async_copy(
                obuf.at[slot], o_hbm.at[pl.ds((k - NS) * QT, QT)], sems.at[slot]
            ).wait()

        for b in range(0, QT, CH):
            src = slice(q * QT + b, q * QT + b + CH)
            dst = slice(b, b + CH)
            l1 = l1_ref[src][:, None, :]
            l2 = l2_ref[src][:, None, :]
            obuf[slot, dst] = g1m1[src] * l1 + g2m2[src] * l2

        pltpu.make_async_copy(
            obuf.at[slot], o_hbm.at[pl.ds(k * QT, QT)], sems.at[slot]
        ).start()

    @pl.when(i == NSTEPS - 1)
    def _():
        for kk in range(NCHUNKS - NS, NCHUNKS):
            pltpu.make_async_copy(
                obuf.at[kk % NS], o_hbm.at[pl.ds(kk * QT, QT)], sems.at[kk % NS]
            ).wait()


def kernel(gates1_s, locations1_sc, gates2_s, locations2_sc, mask1_float, mask2_float):
    g1 = gates1_s.reshape(S, 1)
    g2 = gates2_s.reshape(S, 1)
    return pl.pallas_call(
        _body,
        grid=(NSTEPS,),
        in_specs=[
            pl.BlockSpec((CT, 1), lambda i: (i, 0)),
            pl.BlockSpec((CT, C), lambda i: (i, 0)),
            pl.BlockSpec((CT, 1), lambda i: (i, 0)),
            pl.BlockSpec((CT, C), lambda i: (i, 0)),
            pl.BlockSpec((CT, E), lambda i: (i, 0)),
            pl.BlockSpec((CT, E), lambda i: (i, 0)),
        ],
        out_specs=pl.BlockSpec(memory_space=pl.ANY),
        out_shape=jax.ShapeDtypeStruct((S, E, C), jnp.float32),
        scratch_shapes=[
            pltpu.VMEM((NS, QT, E, C), jnp.float32),
            pltpu.SemaphoreType.DMA((NS,)),
        ],
    )(g1, locations1_sc, g2, locations2_sc, mask1_float, mask2_float)
